# baseline (speedup 1.0000x reference)
import functools
import math

import jax
import jax.numpy as jnp
from jax import lax
from jax.experimental import pallas as pl
from jax.experimental.pallas import tpu as pltpu
from jax.experimental.pallas import tpu_sc as plsc

_N = 512
_F = 64
_H = 64
_E = 8192
_NN = _N * _N
_K = 32
_BI = 128
_BJ = 128
_NBLK = 2

_NW = 32
_EPT = _E // _NW
_ZPT = _NN // 16


def _mm(a, b):
    dims = (((a.ndim - 1,), (0,)), ((), ()))
    return lax.dot_general(a, b, dims, precision=lax.Precision.HIGHEST,
                           preferred_element_type=jnp.float32)


def _sc_body(ei_hbm, ew_hbm, z_hbm, out_hbm, rowv, colv, wvv, idxb, valb, accw):
    cid = lax.axis_index("c")
    sid = lax.axis_index("s")
    wid = sid * 2 + cid
    eb = wid * _EPT
    zb = sid * _ZPT
    pltpu.sync_copy(z_hbm.at[pl.ds(zb, _ZPT)], accw.at[pl.ds(zb, _ZPT)])
    pltpu.sync_copy(ei_hbm.at[0, pl.ds(eb, _EPT)], rowv)
    pltpu.sync_copy(ei_hbm.at[1, pl.ds(eb, _EPT)], colv)
    pltpu.sync_copy(ew_hbm.at[pl.ds(eb, _EPT)], wvv)
    for i in range(_EPT // 16):
        j, off = divmod(i, 8)
        c = colv[pl.ds(i * 16, 16)]
        r = rowv[pl.ds(i * 16, 16)]
        idxb[j, pl.ds(off * 16, 16)] = c * _N + r
        valb[j, pl.ds(off * 16, 16)] = wvv[pl.ds(i * 16, 16)]
    plsc.subcore_barrier()
    for j in range(2):
        pltpu.sync_copy(valb.at[j], accw.at[idxb.at[j]], add=True)
    plsc.subcore_barrier()
    pltpu.sync_copy(accw.at[pl.ds(zb, _ZPT)], out_hbm.at[cid, pl.ds(zb, _ZPT)])


def _build_w(edge_index, edge_weight, zeros):
    mesh = plsc.VectorSubcoreMesh(core_axis_name="c", subcore_axis_name="s")
    return pl.kernel(
        _sc_body,
        out_type=jax.ShapeDtypeStruct((2, _NN), jnp.float32),
        mesh=mesh,
        scratch_types=[
            pltpu.VMEM((_EPT,), jnp.int32),
            pltpu.VMEM((_EPT,), jnp.int32),
            pltpu.VMEM((_EPT,), jnp.float32),
            pltpu.VMEM((2, 128), jnp.int32),
            pltpu.VMEM((2, 128), jnp.float32),
            pltpu.VMEM_SHARED((_NN,), jnp.float32),
        ],
    )(edge_index, edge_weight, zeros)


def _tconv(h, w1, b1, w2, b2, w3, b3):
    p = _mm(h, w1[...]) + b1[...]
    q = jax.nn.sigmoid(_mm(h, w2[...]) + b2[...])
    r = _mm(h, w3[...]) + b3[...]
    return jnp.maximum(p * q + r, 0.0)


def _gnn_body(bsz, *refs):
    it = iter(refs)
    wp = next(it)
    x = next(it)
    inw = next(it)
    inb = next(it)
    blocks = []
    for _ in range(_NBLK):
        blocks.append([next(it) for _ in range(19)])
    emb_ref = next(it)
    embt_ref = next(it)

    wm = wp[0] + wp[1]
    ones_col = jnp.ones((_N, 1), jnp.float32)
    deg = lax.dot_general(wm, ones_col, (((0,), (0,)), ((), ())),
                          precision=lax.Precision.HIGHEST,
                          preferred_element_type=jnp.float32)
    pos = deg > 0.0
    dinv = jnp.where(pos, 1.0 / jnp.sqrt(jnp.where(pos, deg, 1.0)), 0.0)
    dinv_s = jnp.concatenate([dinv] * bsz, axis=0)

    xs = jnp.concatenate([x[b] for b in range(bsz)], axis=0)
    h = _mm(xs, inw[...]) + inb[...]
    for blk in blocks:
        (t1w1, t1b1, t1w2, t1b2, t1w3, t1b3,
         cw0, cw1, cb,
         t2w1, t2b1, t2w2, t2b2, t2w3, t2b3,
         bng, bnb, lng, lnb) = blk
        t0 = _tconv(h, t1w1, t1b1, t1w2, t1b2, t1w3, t1b3)
        u = t0 * dinv_s
        ucat = jnp.concatenate(
            [u[b * _N:(b + 1) * _N] for b in range(bsz)], axis=1)
        ycat = _mm(wm, ucat)
        sx = -(dinv_s * jnp.concatenate(
            [ycat[:, b * _H:(b + 1) * _H] for b in range(bsz)], axis=0))
        t1 = jnp.maximum(_mm(t0, cw0[...]) + _mm(sx, cw1[...]) + cb[...], 0.0)
        t2 = _tconv(t1, t2w1, t2b1, t2w2, t2b2, t2w3, t2b3)
        bscale = bng[...] * (1.0 / math.sqrt(1.0 + 1e-5))
        t2 = t2 * jnp.concatenate([bscale] * bsz, axis=0) \
            + jnp.concatenate([bnb[...]] * bsz, axis=0)
        m = jnp.mean(t2, axis=-1, keepdims=True)
        v = jnp.mean((t2 - m) ** 2, axis=-1, keepdims=True)
        t2 = (t2 - m) / jnp.sqrt(v + 1e-5) * lng[...] + lnb[...]
        h = h + t2

    rr = lax.broadcasted_iota(jnp.int32, (_H, _H), 0)
    cc = lax.broadcasted_iota(jnp.int32, (_H, _H), 1)
    eye = (rr == cc).astype(jnp.float32)
    for b in range(bsz):
        hb = h[b * _N:(b + 1) * _N]
        emb_ref[b] = hb
        embt_ref[b] = lax.dot_general(eye, hb, (((1,), (1,)), ((), ())),
                                      precision=lax.Precision.HIGHEST,
                                      preferred_element_type=jnp.float32)


def _gnn(bsz, wp, xe, flat):
    def full(a):
        nd = a.ndim
        return pl.BlockSpec(a.shape, lambda _n=nd: (0,) * _n)

    in_specs = [full(wp), full(xe)]
    in_specs += [full(a) for a in flat]
    out_specs = [pl.BlockSpec((bsz, _N, _H), lambda: (0, 0, 0)),
                 pl.BlockSpec((bsz, _H, _N), lambda: (0, 0, 0))]
    return pl.pallas_call(
        functools.partial(_gnn_body, bsz),
        in_specs=in_specs,
        out_specs=out_specs,
        out_shape=[jax.ShapeDtypeStruct((bsz, _N, _H), jnp.float32),
                   jax.ShapeDtypeStruct((bsz, _H, _N), jnp.float32)],
    )(wp, xe, *flat)


def _dec_body(emb, embt, wa, wbt, o1b, olng, olnb, o2w, o2b, out_ref):
    e = emb[0]
    et = embt[0]
    a = _mm(e, wa[...]) + o1b[...]
    bt = _mm(wbt[...], et)
    s1 = jnp.zeros((_BI, _BJ), jnp.float32)
    s2 = jnp.zeros((_BI, _BJ), jnp.float32)
    sg = jnp.zeros((_BI, _BJ), jnp.float32)
    gtot = 0.0
    cb = 0.0
    for k in range(_K):
        zk = jnp.maximum(a[:, k:k + 1] + bt[k:k + 1, :], 0.0)
        gk = olng[0, k] * o2w[0, k]
        s1 = s1 + zk
        s2 = s2 + zk * zk
        sg = sg + gk * zk
        gtot = gtot + gk
        cb = cb + olnb[0, k] * o2w[0, k]
    m = s1 * (1.0 / _K)
    v = s2 * (1.0 / _K) - m * m
    inv = 1.0 / jnp.sqrt(v + 1e-5)
    out_ref[0] = jax.nn.sigmoid((sg - gtot * m) * inv + (cb + o2b[0, 0]))


def _decode(bsz, emb, embt, wa, wbt, o1b, olng, olnb, o2w, o2b):
    smem = pltpu.MemorySpace.SMEM
    in_specs = [
        pl.BlockSpec((1, _BI, _H), lambda b, i, j: (b, i, 0)),
        pl.BlockSpec((1, _H, _BJ), lambda b, i, j: (b, 0, j)),
        pl.BlockSpec((_H, _K), lambda b, i, j: (0, 0)),
        pl.BlockSpec((_K, _H), lambda b, i, j: (0, 0)),
        pl.BlockSpec((1, _K), lambda b, i, j: (0, 0)),
        pl.BlockSpec(memory_space=smem),
        pl.BlockSpec(memory_space=smem),
        pl.BlockSpec(memory_space=smem),
        pl.BlockSpec(memory_space=smem),
    ]
    return pl.pallas_call(
        _dec_body,
        grid=(bsz, _N // _BI, _N // _BJ),
        in_specs=in_specs,
        out_specs=pl.BlockSpec((1, _BI, _BJ), lambda b, i, j: (b, i, j)),
        out_shape=jax.ShapeDtypeStruct((bsz, _N, _N), jnp.float32),
    )(emb, embt, wa, wbt, o1b, olng, olnb, o2w, o2b)


def _flatten_params(p):
    flat = [p['in_w'], p['in_b'].reshape(1, _H)]
    for blk in p['blocks']:
        flat += [
            blk['t1c1_w'], blk['t1c1_b'].reshape(1, _H),
            blk['t1c2_w'], blk['t1c2_b'].reshape(1, _H),
            blk['t1c3_w'], blk['t1c3_b'].reshape(1, _H),
            blk['cheb_w0'], blk['cheb_w1'], blk['cheb_b'].reshape(1, _H),
            blk['t2c1_w'], blk['t2c1_b'].reshape(1, _H),
            blk['t2c2_w'], blk['t2c2_b'].reshape(1, _H),
            blk['t2c3_w'], blk['t2c3_b'].reshape(1, _H),
            blk['bn_g'].reshape(_N, 1), blk['bn_b'].reshape(_N, 1),
            blk['ln_g'].reshape(1, _H), blk['ln_b'].reshape(1, _H),
        ]
    return flat


def kernel(x, edge_index, edge_weight, params):
    bsz = x.shape[0]
    xe = x[:, -1]
    zeros = jnp.zeros((_NN,), jnp.float32)
    wpart = _build_w(edge_index, edge_weight, zeros)
    wp = wpart.reshape(2, _N, _N)
    emb, embt = _gnn(bsz, wp, xe, _flatten_params(params))
    p = params
    return _decode(
        bsz, emb, embt,
        p['o1_w'][:_H],
        p['o1_w'][_H:].T,
        p['o1_b'].reshape(1, _K),
        p['oln_g'].reshape(1, _K),
        p['oln_b'].reshape(1, _K),
        p['o2_w'].reshape(1, _K),
        p['o2_b'].reshape(1, 1),
    )


# speedup vs baseline: 1.1839x; 1.1839x over previous
import functools
import math

import jax
import jax.numpy as jnp
from jax import lax
from jax.experimental import pallas as pl
from jax.experimental.pallas import tpu as pltpu
from jax.experimental.pallas import tpu_sc as plsc

_N = 512
_F = 64
_H = 64
_E = 8192
_NN = _N * _N
_K = 32
_BI = 128
_BJ = 128
_NBLK = 2

_NW = 32
_EPT = _E // _NW
_ZPT = _NN // 16


def _mm(a, b):
    dims = (((a.ndim - 1,), (0,)), ((), ()))
    return lax.dot_general(a, b, dims, precision=lax.Precision.HIGHEST,
                           preferred_element_type=jnp.float32)


def _sc_body(ei_hbm, ew_hbm, z_hbm, out_hbm, rowv, colv, wvv, idxb, valb, accw):
    cid = lax.axis_index("c")
    sid = lax.axis_index("s")
    wid = sid * 2 + cid
    eb = wid * _EPT
    zb = sid * _ZPT
    pltpu.sync_copy(z_hbm.at[pl.ds(zb, _ZPT)], accw.at[pl.ds(zb, _ZPT)])
    pltpu.sync_copy(ei_hbm.at[0, pl.ds(eb, _EPT)], rowv)
    pltpu.sync_copy(ei_hbm.at[1, pl.ds(eb, _EPT)], colv)
    pltpu.sync_copy(ew_hbm.at[pl.ds(eb, _EPT)], wvv)
    for i in range(_EPT // 16):
        j, off = divmod(i, 8)
        c = colv[pl.ds(i * 16, 16)]
        r = rowv[pl.ds(i * 16, 16)]
        idxb[j, pl.ds(off * 16, 16)] = c * _N + r
        valb[j, pl.ds(off * 16, 16)] = wvv[pl.ds(i * 16, 16)]
    plsc.subcore_barrier()
    for j in range(2):
        pltpu.sync_copy(valb.at[j], accw.at[idxb.at[j]], add=True)
    plsc.subcore_barrier()
    pltpu.sync_copy(accw.at[pl.ds(zb, _ZPT)], out_hbm.at[cid, pl.ds(zb, _ZPT)])


def _build_w(edge_index, edge_weight, zeros):
    mesh = plsc.VectorSubcoreMesh(core_axis_name="c", subcore_axis_name="s")
    return pl.kernel(
        _sc_body,
        out_type=jax.ShapeDtypeStruct((2, _NN), jnp.float32),
        mesh=mesh,
        scratch_types=[
            pltpu.VMEM((_EPT,), jnp.int32),
            pltpu.VMEM((_EPT,), jnp.int32),
            pltpu.VMEM((_EPT,), jnp.float32),
            pltpu.VMEM((2, 128), jnp.int32),
            pltpu.VMEM((2, 128), jnp.float32),
            pltpu.VMEM_SHARED((_NN,), jnp.float32),
        ],
    )(edge_index, edge_weight, zeros)


def _tconv(h, w1, b1, w2, b2, w3, b3):
    p = _mm(h, w1[...]) + b1[...]
    q = jax.nn.sigmoid(_mm(h, w2[...]) + b2[...])
    r = _mm(h, w3[...]) + b3[...]
    return jnp.maximum(p * q + r, 0.0)


def _gnn_body(bsz, *refs):
    it = iter(refs)
    wp = next(it)
    x = next(it)
    inw = next(it)
    inb = next(it)
    blocks = []
    for _ in range(_NBLK):
        blocks.append([next(it) for _ in range(19)])
    emb_ref = next(it)
    embt_ref = next(it)

    wm = wp[0] + wp[1]
    ones_col = jnp.ones((_N, 1), jnp.float32)
    deg = lax.dot_general(wm, ones_col, (((0,), (0,)), ((), ())),
                          precision=lax.Precision.HIGHEST,
                          preferred_element_type=jnp.float32)
    pos = deg > 0.0
    dinv = jnp.where(pos, 1.0 / jnp.sqrt(jnp.where(pos, deg, 1.0)), 0.0)
    dinv_s = jnp.concatenate([dinv] * bsz, axis=0)

    xs = jnp.concatenate([x[b] for b in range(bsz)], axis=0)
    h = _mm(xs, inw[...]) + inb[...]
    for blk in blocks:
        (t1w1, t1b1, t1w2, t1b2, t1w3, t1b3,
         cw0, cw1, cb,
         t2w1, t2b1, t2w2, t2b2, t2w3, t2b3,
         bng, bnb, lng, lnb) = blk
        t0 = _tconv(h, t1w1, t1b1, t1w2, t1b2, t1w3, t1b3)
        u = t0 * dinv_s
        ucat = jnp.concatenate(
            [u[b * _N:(b + 1) * _N] for b in range(bsz)], axis=1)
        ycat = _mm(wm, ucat)
        sx = -(dinv_s * jnp.concatenate(
            [ycat[:, b * _H:(b + 1) * _H] for b in range(bsz)], axis=0))
        t1 = jnp.maximum(_mm(t0, cw0[...]) + _mm(sx, cw1[...]) + cb[...], 0.0)
        t2 = _tconv(t1, t2w1, t2b1, t2w2, t2b2, t2w3, t2b3)
        bscale = bng[...] * (1.0 / math.sqrt(1.0 + 1e-5))
        t2 = t2 * jnp.concatenate([bscale] * bsz, axis=0) \
            + jnp.concatenate([bnb[...]] * bsz, axis=0)
        m = jnp.mean(t2, axis=-1, keepdims=True)
        v = jnp.mean((t2 - m) ** 2, axis=-1, keepdims=True)
        t2 = (t2 - m) / jnp.sqrt(v + 1e-5) * lng[...] + lnb[...]
        h = h + t2

    rr = lax.broadcasted_iota(jnp.int32, (_H, _H), 0)
    cc = lax.broadcasted_iota(jnp.int32, (_H, _H), 1)
    eye = (rr == cc).astype(jnp.float32)
    for b in range(bsz):
        hb = h[b * _N:(b + 1) * _N]
        emb_ref[b] = hb
        embt_ref[b] = lax.dot_general(eye, hb, (((1,), (1,)), ((), ())),
                                      precision=lax.Precision.HIGHEST,
                                      preferred_element_type=jnp.float32)


def _gnn(bsz, wp, xe, flat):
    def full(a):
        nd = a.ndim
        return pl.BlockSpec(a.shape, lambda _n=nd: (0,) * _n)

    in_specs = [full(wp), full(xe)]
    in_specs += [full(a) for a in flat]
    out_specs = [pl.BlockSpec((bsz, _N, _H), lambda: (0, 0, 0)),
                 pl.BlockSpec((bsz, _H, _N), lambda: (0, 0, 0))]
    return pl.pallas_call(
        functools.partial(_gnn_body, bsz),
        in_specs=in_specs,
        out_specs=out_specs,
        out_shape=[jax.ShapeDtypeStruct((bsz, _N, _H), jnp.float32),
                   jax.ShapeDtypeStruct((bsz, _H, _N), jnp.float32)],
    )(wp, xe, *flat)


def _dec_body(emb, embt, wa, wbt, o1b, olng, olnb, o2w, o2b, out_ref):
    e = emb[0]
    et = embt[0]
    a = _mm(e, wa[...]) + o1b[...]
    bt = _mm(wbt[...], et)
    gs = [olng[0, k] * o2w[0, k] for k in range(_K)]
    gtot = 0.0
    cb = 0.0
    for k in range(_K):
        gtot = gtot + gs[k]
        cb = cb + olnb[0, k] * o2w[0, k]
    cb = cb + o2b[0, 0]
    for jc in range(_N // _BJ):
        btc = bt[:, jc * _BJ:(jc + 1) * _BJ]
        s1 = jnp.zeros((_BI, _BJ), jnp.float32)
        s2 = jnp.zeros((_BI, _BJ), jnp.float32)
        sg = jnp.zeros((_BI, _BJ), jnp.float32)
        for k in range(_K):
            zk = jnp.maximum(a[:, k:k + 1] + btc[k:k + 1, :], 0.0)
            s1 = s1 + zk
            s2 = s2 + zk * zk
            sg = sg + gs[k] * zk
        m = s1 * (1.0 / _K)
        v = s2 * (1.0 / _K) - m * m
        inv = 1.0 / jnp.sqrt(v + 1e-5)
        out_ref[0, :, jc * _BJ:(jc + 1) * _BJ] = jax.nn.sigmoid(
            (sg - gtot * m) * inv + cb)


def _decode(bsz, emb, embt, wa, wbt, o1b, olng, olnb, o2w, o2b):
    smem = pltpu.MemorySpace.SMEM
    in_specs = [
        pl.BlockSpec((1, _BI, _H), lambda b, i: (b, i, 0)),
        pl.BlockSpec((1, _H, _N), lambda b, i: (b, 0, 0)),
        pl.BlockSpec((_H, _K), lambda b, i: (0, 0)),
        pl.BlockSpec((_K, _H), lambda b, i: (0, 0)),
        pl.BlockSpec((1, _K), lambda b, i: (0, 0)),
        pl.BlockSpec(memory_space=smem),
        pl.BlockSpec(memory_space=smem),
        pl.BlockSpec(memory_space=smem),
        pl.BlockSpec(memory_space=smem),
    ]
    return pl.pallas_call(
        _dec_body,
        grid=(bsz, _N // _BI),
        in_specs=in_specs,
        out_specs=pl.BlockSpec((1, _BI, _N), lambda b, i: (b, i, 0)),
        out_shape=jax.ShapeDtypeStruct((bsz, _N, _N), jnp.float32),
    )(emb, embt, wa, wbt, o1b, olng, olnb, o2w, o2b)


def _flatten_params(p):
    flat = [p['in_w'], p['in_b'].reshape(1, _H)]
    for blk in p['blocks']:
        flat += [
            blk['t1c1_w'], blk['t1c1_b'].reshape(1, _H),
            blk['t1c2_w'], blk['t1c2_b'].reshape(1, _H),
            blk['t1c3_w'], blk['t1c3_b'].reshape(1, _H),
            blk['cheb_w0'], blk['cheb_w1'], blk['cheb_b'].reshape(1, _H),
            blk['t2c1_w'], blk['t2c1_b'].reshape(1, _H),
            blk['t2c2_w'], blk['t2c2_b'].reshape(1, _H),
            blk['t2c3_w'], blk['t2c3_b'].reshape(1, _H),
            blk['bn_g'].reshape(_N, 1), blk['bn_b'].reshape(_N, 1),
            blk['ln_g'].reshape(1, _H), blk['ln_b'].reshape(1, _H),
        ]
    return flat


def kernel(x, edge_index, edge_weight, params):
    bsz = x.shape[0]
    xe = x[:, -1]
    zeros = jnp.zeros((_NN,), jnp.float32)
    wpart = _build_w(edge_index, edge_weight, zeros)
    wp = wpart.reshape(2, _N, _N)
    emb, embt = _gnn(bsz, wp, xe, _flatten_params(params))
    p = params
    return _decode(
        bsz, emb, embt,
        p['o1_w'][:_H],
        p['o1_w'][_H:].T,
        p['o1_b'].reshape(1, _K),
        p['oln_g'].reshape(1, _K),
        p['oln_b'].reshape(1, _K),
        p['o2_w'].reshape(1, _K),
        p['o2_b'].reshape(1, 1),
    )


# speedup vs baseline: 1.3241x; 1.1185x over previous
import functools
import math

import jax
import jax.numpy as jnp
from jax import lax
from jax.experimental import pallas as pl
from jax.experimental.pallas import tpu as pltpu
from jax.experimental.pallas import tpu_sc as plsc

_N = 512
_F = 64
_H = 64
_E = 8192
_NN = _N * _N
_K = 32
_BI = 128
_BJ = 128
_NBLK = 2

_NW = 32
_EPT = _E // _NW
_ZPT = _NN // 16


def _mm(a, b):
    dims = (((a.ndim - 1,), (0,)), ((), ()))
    return lax.dot_general(a, b, dims, preferred_element_type=jnp.float32)


def _sc_body(ei_hbm, ew_hbm, z_hbm, out_hbm, rowv, colv, wvv, idxb, valb, accw):
    cid = lax.axis_index("c")
    sid = lax.axis_index("s")
    wid = sid * 2 + cid
    eb = wid * _EPT
    zb = sid * _ZPT
    pltpu.sync_copy(z_hbm.at[pl.ds(zb, _ZPT)], accw.at[pl.ds(zb, _ZPT)])
    pltpu.sync_copy(ei_hbm.at[0, pl.ds(eb, _EPT)], rowv)
    pltpu.sync_copy(ei_hbm.at[1, pl.ds(eb, _EPT)], colv)
    pltpu.sync_copy(ew_hbm.at[pl.ds(eb, _EPT)], wvv)
    for i in range(_EPT // 16):
        j, off = divmod(i, 8)
        c = colv[pl.ds(i * 16, 16)]
        r = rowv[pl.ds(i * 16, 16)]
        idxb[j, pl.ds(off * 16, 16)] = c * _N + r
        valb[j, pl.ds(off * 16, 16)] = wvv[pl.ds(i * 16, 16)]
    plsc.subcore_barrier()
    for j in range(2):
        pltpu.sync_copy(valb.at[j], accw.at[idxb.at[j]], add=True)
    plsc.subcore_barrier()
    pltpu.sync_copy(accw.at[pl.ds(zb, _ZPT)], out_hbm.at[cid, pl.ds(zb, _ZPT)])


def _build_w(edge_index, edge_weight, zeros):
    mesh = plsc.VectorSubcoreMesh(core_axis_name="c", subcore_axis_name="s")
    return pl.kernel(
        _sc_body,
        out_type=jax.ShapeDtypeStruct((2, _NN), jnp.float32),
        mesh=mesh,
        scratch_types=[
            pltpu.VMEM((_EPT,), jnp.int32),
            pltpu.VMEM((_EPT,), jnp.int32),
            pltpu.VMEM((_EPT,), jnp.float32),
            pltpu.VMEM((2, 128), jnp.int32),
            pltpu.VMEM((2, 128), jnp.float32),
            pltpu.VMEM_SHARED((_NN,), jnp.float32),
        ],
    )(edge_index, edge_weight, zeros)


def _tconv(h, w1, b1, w2, b2, w3, b3):
    p = _mm(h, w1[...]) + b1[...]
    q = jax.nn.sigmoid(_mm(h, w2[...]) + b2[...])
    r = _mm(h, w3[...]) + b3[...]
    return jnp.maximum(p * q + r, 0.0)


def _gnn_body(bsz, *refs):
    it = iter(refs)
    wp = next(it)
    x = next(it)
    inw = next(it)
    inb = next(it)
    blocks = []
    for _ in range(_NBLK):
        blocks.append([next(it) for _ in range(19)])
    emb_ref = next(it)
    embt_ref = next(it)

    wm = wp[0] + wp[1]
    ones_col = jnp.ones((_N, 1), jnp.float32)
    deg = lax.dot_general(wm, ones_col, (((0,), (0,)), ((), ())),
                          precision=lax.Precision.HIGHEST,
                          preferred_element_type=jnp.float32)
    pos = deg > 0.0
    dinv = jnp.where(pos, 1.0 / jnp.sqrt(jnp.where(pos, deg, 1.0)), 0.0)
    dinv_s = jnp.concatenate([dinv] * bsz, axis=0)

    xs = jnp.concatenate([x[b, 0] for b in range(bsz)], axis=0)
    h = _mm(xs, inw[...]) + inb[...]
    for blk in blocks:
        (t1w1, t1b1, t1w2, t1b2, t1w3, t1b3,
         cw0, cw1, cb,
         t2w1, t2b1, t2w2, t2b2, t2w3, t2b3,
         bng, bnb, lng, lnb) = blk
        t0 = _tconv(h, t1w1, t1b1, t1w2, t1b2, t1w3, t1b3)
        u = t0 * dinv_s
        ucat = jnp.concatenate(
            [u[b * _N:(b + 1) * _N] for b in range(bsz)], axis=1)
        ycat = _mm(wm, ucat)
        sx = -(dinv_s * jnp.concatenate(
            [ycat[:, b * _H:(b + 1) * _H] for b in range(bsz)], axis=0))
        t1 = jnp.maximum(_mm(t0, cw0[...]) + _mm(sx, cw1[...]) + cb[...], 0.0)
        t2 = _tconv(t1, t2w1, t2b1, t2w2, t2b2, t2w3, t2b3)
        bscale = bng[...] * (1.0 / math.sqrt(1.0 + 1e-5))
        t2 = t2 * jnp.concatenate([bscale] * bsz, axis=0) \
            + jnp.concatenate([bnb[...]] * bsz, axis=0)
        m = jnp.mean(t2, axis=-1, keepdims=True)
        v = jnp.mean((t2 - m) ** 2, axis=-1, keepdims=True)
        t2 = (t2 - m) / jnp.sqrt(v + 1e-5) * lng[...] + lnb[...]
        h = h + t2

    rr = lax.broadcasted_iota(jnp.int32, (_H, _H), 0)
    cc = lax.broadcasted_iota(jnp.int32, (_H, _H), 1)
    eye = (rr == cc).astype(jnp.float32)
    for b in range(bsz):
        hb = h[b * _N:(b + 1) * _N]
        emb_ref[b] = hb
        embt_ref[b] = lax.dot_general(eye, hb, (((1,), (1,)), ((), ())),
                                      precision=lax.Precision.HIGHEST,
                                      preferred_element_type=jnp.float32)


def _gnn(bsz, wp, x, flat):
    def full(a):
        nd = a.ndim
        return pl.BlockSpec(a.shape, lambda g, _n=nd: (0,) * _n)

    tlast = x.shape[1] - 1
    in_specs = [full(wp),
                pl.BlockSpec((bsz, 1, _N, _F), lambda g: (0, tlast, 0, 0))]
    in_specs += [full(a) for a in flat]
    out_specs = [pl.BlockSpec((bsz, _N, _H), lambda g: (0, 0, 0)),
                 pl.BlockSpec((bsz, _H, _N), lambda g: (0, 0, 0))]
    return pl.pallas_call(
        functools.partial(_gnn_body, bsz),
        grid=(1,),
        in_specs=in_specs,
        out_specs=out_specs,
        out_shape=[jax.ShapeDtypeStruct((bsz, _N, _H), jnp.float32),
                   jax.ShapeDtypeStruct((bsz, _H, _N), jnp.float32)],
    )(wp, x, *flat)


def _dec_body(emb, embt, wa, wbt, o1b, olng, olnb, o2w, o2b, out_ref):
    e = emb[0]
    et = embt[0]
    a = _mm(e, wa[...]) + o1b[...]
    bt = _mm(wbt[...], et)
    gs = [olng[k] * o2w[k, 0] for k in range(_K)]
    gtot = 0.0
    cb = 0.0
    for k in range(_K):
        gtot = gtot + gs[k]
        cb = cb + olnb[k] * o2w[k, 0]
    cb = cb + o2b[0]
    for jc in range(_N // _BJ):
        btc = bt[:, jc * _BJ:(jc + 1) * _BJ]
        s1 = jnp.zeros((_BI, _BJ), jnp.float32)
        s2 = jnp.zeros((_BI, _BJ), jnp.float32)
        sg = jnp.zeros((_BI, _BJ), jnp.float32)
        for k in range(_K):
            zk = jnp.maximum(a[:, k:k + 1] + btc[k:k + 1, :], 0.0)
            s1 = s1 + zk
            s2 = s2 + zk * zk
            sg = sg + gs[k] * zk
        m = s1 * (1.0 / _K)
        v = s2 * (1.0 / _K) - m * m
        inv = 1.0 / jnp.sqrt(v + 1e-5)
        out_ref[0, :, jc * _BJ:(jc + 1) * _BJ] = jax.nn.sigmoid(
            (sg - gtot * m) * inv + cb)


def _decode(bsz, emb, embt, wa, wbt, o1b, olng, olnb, o2w, o2b):
    smem = pltpu.MemorySpace.SMEM
    in_specs = [
        pl.BlockSpec((1, _BI, _H), lambda b, i: (b, i, 0)),
        pl.BlockSpec((1, _H, _N), lambda b, i: (b, 0, 0)),
        pl.BlockSpec((_H, _K), lambda b, i: (0, 0)),
        pl.BlockSpec((_K, _H), lambda b, i: (0, 0)),
        pl.BlockSpec((_K,), lambda b, i: (0,)),
        pl.BlockSpec(memory_space=smem),
        pl.BlockSpec(memory_space=smem),
        pl.BlockSpec(memory_space=smem),
        pl.BlockSpec(memory_space=smem),
    ]
    return pl.pallas_call(
        _dec_body,
        grid=(bsz, _N // _BI),
        in_specs=in_specs,
        out_specs=pl.BlockSpec((1, _BI, _N), lambda b, i: (b, i, 0)),
        out_shape=jax.ShapeDtypeStruct((bsz, _N, _N), jnp.float32),
    )(emb, embt, wa, wbt, o1b, olng, olnb, o2w, o2b)


def _flatten_params(p):
    flat = [p['in_w'], p['in_b']]
    for blk in p['blocks']:
        flat += [
            blk['t1c1_w'], blk['t1c1_b'],
            blk['t1c2_w'], blk['t1c2_b'],
            blk['t1c3_w'], blk['t1c3_b'],
            blk['cheb_w0'], blk['cheb_w1'], blk['cheb_b'],
            blk['t2c1_w'], blk['t2c1_b'],
            blk['t2c2_w'], blk['t2c2_b'],
            blk['t2c3_w'], blk['t2c3_b'],
            blk['bn_g'].reshape(_N, 1), blk['bn_b'].reshape(_N, 1),
            blk['ln_g'], blk['ln_b'],
        ]
    return flat


def kernel(x, edge_index, edge_weight, params):
    bsz = x.shape[0]
    zeros = jnp.zeros((_NN,), jnp.float32)
    wpart = _build_w(edge_index, edge_weight, zeros)
    wp = wpart.reshape(2, _N, _N)
    emb, embt = _gnn(bsz, wp, x, _flatten_params(params))
    p = params
    return _decode(
        bsz, emb, embt,
        p['o1_w'][:_H],
        p['o1_w'][_H:].T,
        p['o1_b'],
        p['oln_g'],
        p['oln_b'],
        p['o2_w'],
        p['o2_b'],
    )


# speedup vs baseline: 1.3766x; 1.0396x over previous
import functools
import math

import jax
import jax.numpy as jnp
from jax import lax
from jax.experimental import pallas as pl
from jax.experimental.pallas import tpu as pltpu
from jax.experimental.pallas import tpu_sc as plsc

_N = 512
_F = 64
_H = 64
_E = 8192
_NN = _N * _N
_K = 32
_BI = 128
_BJ = 128
_NBLK = 2

_EPT = _E // 16
_ZPT = _NN // 16


def _mm(a, b):
    dims = (((a.ndim - 1,), (0,)), ((), ()))
    return lax.dot_general(a, b, dims, preferred_element_type=jnp.float32)


def _sc_body(ei_hbm, ew_hbm, out_hbm, rowv, colv, wvv, idxb, valb, zbuf, accw):
    cid = lax.axis_index("c")
    sid = lax.axis_index("s")

    @pl.when(cid == 0)
    def _work():
        eb = sid * _EPT
        zb = sid * _ZPT
        zv = jnp.zeros((16,), jnp.float32)
        for i in range(128):
            zbuf[pl.ds(i * 16, 16)] = zv
        for t in range(_ZPT // 2048):
            pltpu.sync_copy(zbuf, accw.at[pl.ds(zb + t * 2048, 2048)])
        pltpu.sync_copy(ei_hbm.at[0, pl.ds(eb, _EPT)], rowv)
        pltpu.sync_copy(ei_hbm.at[1, pl.ds(eb, _EPT)], colv)
        pltpu.sync_copy(ew_hbm.at[pl.ds(eb, _EPT)], wvv)
        for i in range(_EPT // 16):
            j, off = divmod(i, 8)
            c = colv[pl.ds(i * 16, 16)]
            r = rowv[pl.ds(i * 16, 16)]
            idxb[j, pl.ds(off * 16, 16)] = c * _N + r
            valb[j, pl.ds(off * 16, 16)] = wvv[pl.ds(i * 16, 16)]
        plsc.subcore_barrier()
        for j in range(_EPT // 128):
            pltpu.sync_copy(valb.at[j], accw.at[idxb.at[j]], add=True)
        plsc.subcore_barrier()
        pltpu.sync_copy(accw.at[pl.ds(zb, _ZPT)], out_hbm.at[pl.ds(zb, _ZPT)])


def _build_w(edge_index, edge_weight):
    mesh = plsc.VectorSubcoreMesh(core_axis_name="c", subcore_axis_name="s")
    return pl.kernel(
        _sc_body,
        out_type=jax.ShapeDtypeStruct((_NN,), jnp.float32),
        mesh=mesh,
        scratch_types=[
            pltpu.VMEM((_EPT,), jnp.int32),
            pltpu.VMEM((_EPT,), jnp.int32),
            pltpu.VMEM((_EPT,), jnp.float32),
            pltpu.VMEM((4, 128), jnp.int32),
            pltpu.VMEM((4, 128), jnp.float32),
            pltpu.VMEM((2048,), jnp.float32),
            pltpu.VMEM_SHARED((_NN,), jnp.float32),
        ],
    )(edge_index, edge_weight)


def _tconv(h, w1, b1, w2, b2, w3, b3):
    p = _mm(h, w1[...]) + b1[...]
    q = jax.nn.sigmoid(_mm(h, w2[...]) + b2[...])
    r = _mm(h, w3[...]) + b3[...]
    return jnp.maximum(p * q + r, 0.0)


def _gnn_body(bsz, *refs):
    it = iter(refs)
    wp = next(it)
    x = next(it)
    inw = next(it)
    inb = next(it)
    blocks = []
    for _ in range(_NBLK):
        blocks.append([next(it) for _ in range(19)])
    emb_ref = next(it)
    embt_ref = next(it)

    wm = wp[...]
    ones_col = jnp.ones((_N, 1), jnp.float32)
    deg = lax.dot_general(wm, ones_col, (((0,), (0,)), ((), ())),
                          precision=lax.Precision.HIGHEST,
                          preferred_element_type=jnp.float32)
    pos = deg > 0.0
    dinv = jnp.where(pos, 1.0 / jnp.sqrt(jnp.where(pos, deg, 1.0)), 0.0)
    dinv_s = jnp.concatenate([dinv] * bsz, axis=0)

    xs = jnp.concatenate([x[b, 0] for b in range(bsz)], axis=0)
    h = _mm(xs, inw[...]) + inb[...]
    for blk in blocks:
        (t1w1, t1b1, t1w2, t1b2, t1w3, t1b3,
         cw0, cw1, cb,
         t2w1, t2b1, t2w2, t2b2, t2w3, t2b3,
         bng, bnb, lng, lnb) = blk
        t0 = _tconv(h, t1w1, t1b1, t1w2, t1b2, t1w3, t1b3)
        u = t0 * dinv_s
        ucat = jnp.concatenate(
            [u[b * _N:(b + 1) * _N] for b in range(bsz)], axis=1)
        ycat = _mm(wm, ucat)
        sx = -(dinv_s * jnp.concatenate(
            [ycat[:, b * _H:(b + 1) * _H] for b in range(bsz)], axis=0))
        t1 = jnp.maximum(_mm(t0, cw0[...]) + _mm(sx, cw1[...]) + cb[...], 0.0)
        t2 = _tconv(t1, t2w1, t2b1, t2w2, t2b2, t2w3, t2b3)
        bscale = bng[...] * (1.0 / math.sqrt(1.0 + 1e-5))
        t2 = t2 * jnp.concatenate([bscale] * bsz, axis=0) \
            + jnp.concatenate([bnb[...]] * bsz, axis=0)
        m = jnp.mean(t2, axis=-1, keepdims=True)
        v = jnp.mean((t2 - m) ** 2, axis=-1, keepdims=True)
        t2 = (t2 - m) / jnp.sqrt(v + 1e-5) * lng[...] + lnb[...]
        h = h + t2

    rr = lax.broadcasted_iota(jnp.int32, (_H, _H), 0)
    cc = lax.broadcasted_iota(jnp.int32, (_H, _H), 1)
    eye = (rr == cc).astype(jnp.float32)
    for b in range(bsz):
        hb = h[b * _N:(b + 1) * _N]
        emb_ref[b] = hb
        embt_ref[b] = lax.dot_general(eye, hb, (((1,), (1,)), ((), ())),
                                      precision=lax.Precision.HIGHEST,
                                      preferred_element_type=jnp.float32)


def _gnn(bsz, wp, x, flat):
    def full(a):
        nd = a.ndim
        return pl.BlockSpec(a.shape, lambda g, _n=nd: (0,) * _n)

    tlast = x.shape[1] - 1
    in_specs = [full(wp),
                pl.BlockSpec((bsz, 1, _N, _F), lambda g: (0, tlast, 0, 0))]
    in_specs += [full(a) for a in flat]
    out_specs = [pl.BlockSpec((bsz, _N, _H), lambda g: (0, 0, 0)),
                 pl.BlockSpec((bsz, _H, _N), lambda g: (0, 0, 0))]
    return pl.pallas_call(
        functools.partial(_gnn_body, bsz),
        grid=(1,),
        in_specs=in_specs,
        out_specs=out_specs,
        out_shape=[jax.ShapeDtypeStruct((bsz, _N, _H), jnp.float32),
                   jax.ShapeDtypeStruct((bsz, _H, _N), jnp.float32)],
    )(wp, x, *flat)


def _dec_body(emb, embt, wa, wbt, o1b, olng, olnb, o2w, o2b, out_ref):
    e = emb[0]
    et = embt[0]
    a = _mm(e, wa[...]) + o1b[...]
    bt = _mm(wbt[...], et)
    gs = [olng[k] * o2w[k, 0] for k in range(_K)]
    gtot = 0.0
    cb = 0.0
    for k in range(_K):
        gtot = gtot + gs[k]
        cb = cb + olnb[k] * o2w[k, 0]
    cb = cb + o2b[0]
    for jc in range(_N // _BJ):
        btc = bt[:, jc * _BJ:(jc + 1) * _BJ]
        s1 = jnp.zeros((_BI, _BJ), jnp.float32)
        s2 = jnp.zeros((_BI, _BJ), jnp.float32)
        sg = jnp.zeros((_BI, _BJ), jnp.float32)
        for k in range(_K):
            zk = jnp.maximum(a[:, k:k + 1] + btc[k:k + 1, :], 0.0)
            s1 = s1 + zk
            s2 = s2 + zk * zk
            sg = sg + gs[k] * zk
        m = s1 * (1.0 / _K)
        v = s2 * (1.0 / _K) - m * m
        inv = 1.0 / jnp.sqrt(v + 1e-5)
        out_ref[0, :, jc * _BJ:(jc + 1) * _BJ] = jax.nn.sigmoid(
            (sg - gtot * m) * inv + cb)


def _decode(bsz, emb, embt, wa, wbt, o1b, olng, olnb, o2w, o2b):
    smem = pltpu.MemorySpace.SMEM
    in_specs = [
        pl.BlockSpec((1, _BI, _H), lambda b, i: (b, i, 0)),
        pl.BlockSpec((1, _H, _N), lambda b, i: (b, 0, 0)),
        pl.BlockSpec((_H, _K), lambda b, i: (0, 0)),
        pl.BlockSpec((_K, _H), lambda b, i: (0, 0)),
        pl.BlockSpec((_K,), lambda b, i: (0,)),
        pl.BlockSpec(memory_space=smem),
        pl.BlockSpec(memory_space=smem),
        pl.BlockSpec(memory_space=smem),
        pl.BlockSpec(memory_space=smem),
    ]
    return pl.pallas_call(
        _dec_body,
        grid=(bsz, _N // _BI),
        in_specs=in_specs,
        out_specs=pl.BlockSpec((1, _BI, _N), lambda b, i: (b, i, 0)),
        out_shape=jax.ShapeDtypeStruct((bsz, _N, _N), jnp.float32),
    )(emb, embt, wa, wbt, o1b, olng, olnb, o2w, o2b)


def _flatten_params(p):
    flat = [p['in_w'], p['in_b']]
    for blk in p['blocks']:
        flat += [
            blk['t1c1_w'], blk['t1c1_b'],
            blk['t1c2_w'], blk['t1c2_b'],
            blk['t1c3_w'], blk['t1c3_b'],
            blk['cheb_w0'], blk['cheb_w1'], blk['cheb_b'],
            blk['t2c1_w'], blk['t2c1_b'],
            blk['t2c2_w'], blk['t2c2_b'],
            blk['t2c3_w'], blk['t2c3_b'],
            blk['bn_g'].reshape(_N, 1), blk['bn_b'].reshape(_N, 1),
            blk['ln_g'], blk['ln_b'],
        ]
    return flat


def kernel(x, edge_index, edge_weight, params):
    bsz = x.shape[0]
    wpart = _build_w(edge_index, edge_weight)
    wp = wpart.reshape(_N, _N)
    emb, embt = _gnn(bsz, wp, x, _flatten_params(params))
    p = params
    return _decode(
        bsz, emb, embt,
        p['o1_w'][:_H],
        p['o1_w'][_H:].T,
        p['o1_b'],
        p['oln_g'],
        p['oln_b'],
        p['o2_w'],
        p['o2_b'],
    )


# speedup vs baseline: 1.3830x; 1.0047x over previous
import functools
import math

import jax
import jax.numpy as jnp
from jax import lax
from jax.experimental import pallas as pl
from jax.experimental.pallas import tpu as pltpu
from jax.experimental.pallas import tpu_sc as plsc

_N = 512
_F = 64
_H = 64
_E = 8192
_NN = _N * _N
_K = 32
_BI = 128
_BJ = 128
_NBLK = 2

_EPT = _E // 16
_ZPT = _NN // 16


def _mm(a, b):
    dims = (((a.ndim - 1,), (0,)), ((), ()))
    return lax.dot_general(a, b, dims, preferred_element_type=jnp.float32)


def _sc_body(ei_hbm, ew_hbm, out_hbm, rowv, colv, wvv, idxb, valb, zbuf, accw):
    cid = lax.axis_index("c")
    sid = lax.axis_index("s")

    @pl.when(cid == 0)
    def _work():
        eb = sid * _EPT
        zb = sid * _ZPT
        zv = jnp.zeros((16,), jnp.float32)
        for i in range(128):
            zbuf[pl.ds(i * 16, 16)] = zv
        for t in range(_ZPT // 2048):
            pltpu.sync_copy(zbuf, accw.at[pl.ds(zb + t * 2048, 2048)])
        pltpu.sync_copy(ei_hbm.at[0, pl.ds(eb, _EPT)], rowv)
        pltpu.sync_copy(ei_hbm.at[1, pl.ds(eb, _EPT)], colv)
        pltpu.sync_copy(ew_hbm.at[pl.ds(eb, _EPT)], wvv)
        for i in range(_EPT // 16):
            j, off = divmod(i, 8)
            c = colv[pl.ds(i * 16, 16)]
            r = rowv[pl.ds(i * 16, 16)]
            idxb[j, pl.ds(off * 16, 16)] = c * _N + r
            valb[j, pl.ds(off * 16, 16)] = wvv[pl.ds(i * 16, 16)]
        plsc.subcore_barrier()
        for j in range(_EPT // 128):
            pltpu.sync_copy(valb.at[j], accw.at[idxb.at[j]], add=True)
        plsc.subcore_barrier()
        pltpu.sync_copy(accw.at[pl.ds(zb, _ZPT)], out_hbm.at[pl.ds(zb, _ZPT)])


def _build_w(edge_index, edge_weight):
    mesh = plsc.VectorSubcoreMesh(core_axis_name="c", subcore_axis_name="s")
    return pl.kernel(
        _sc_body,
        out_type=jax.ShapeDtypeStruct((_NN,), jnp.float32),
        mesh=mesh,
        scratch_types=[
            pltpu.VMEM((_EPT,), jnp.int32),
            pltpu.VMEM((_EPT,), jnp.int32),
            pltpu.VMEM((_EPT,), jnp.float32),
            pltpu.VMEM((4, 128), jnp.int32),
            pltpu.VMEM((4, 128), jnp.float32),
            pltpu.VMEM((2048,), jnp.float32),
            pltpu.VMEM_SHARED((_NN,), jnp.float32),
        ],
    )(edge_index, edge_weight)


def _tconv(h, w1, b1, w2, b2, w3, b3):
    p = _mm(h, w1[...]) + b1[...]
    q = jax.nn.sigmoid(_mm(h, w2[...]) + b2[...])
    r = _mm(h, w3[...]) + b3[...]
    return jnp.maximum(p * q + r, 0.0)


def _gnn_body(bsz, *refs):
    it = iter(refs)
    wp = next(it)
    x = next(it)
    inw = next(it)
    inb = next(it)
    blocks = []
    for _ in range(_NBLK):
        blocks.append([next(it) for _ in range(19)])
    emb_ref = next(it)
    embt_ref = next(it)

    wm = wp[...]
    ones_col = jnp.ones((_N, 1), jnp.float32)
    deg = lax.dot_general(wm, ones_col, (((0,), (0,)), ((), ())),
                          precision=lax.Precision.HIGHEST,
                          preferred_element_type=jnp.float32)
    pos = deg > 0.0
    dinv = jnp.where(pos, 1.0 / jnp.sqrt(jnp.where(pos, deg, 1.0)), 0.0)
    dinv_s = jnp.concatenate([dinv] * bsz, axis=0)

    xs = jnp.concatenate([x[b, 0] for b in range(bsz)], axis=0)
    h = _mm(xs, inw[...]) + inb[...]
    for blk in blocks:
        (t1w1, t1b1, t1w2, t1b2, t1w3, t1b3,
         cw0, cw1, cb,
         t2w1, t2b1, t2w2, t2b2, t2w3, t2b3,
         bng, bnb, lng, lnb) = blk
        t0 = _tconv(h, t1w1, t1b1, t1w2, t1b2, t1w3, t1b3)
        u = t0 * dinv_s
        ucat = jnp.concatenate(
            [u[b * _N:(b + 1) * _N] for b in range(bsz)], axis=1)
        ycat = _mm(wm, ucat)
        sx = -(dinv_s * jnp.concatenate(
            [ycat[:, b * _H:(b + 1) * _H] for b in range(bsz)], axis=0))
        t1 = jnp.maximum(_mm(t0, cw0[...]) + _mm(sx, cw1[...]) + cb[...], 0.0)
        t2 = _tconv(t1, t2w1, t2b1, t2w2, t2b2, t2w3, t2b3)
        bscale = bng[...] * (1.0 / math.sqrt(1.0 + 1e-5))
        t2 = t2 * jnp.concatenate([bscale] * bsz, axis=0) \
            + jnp.concatenate([bnb[...]] * bsz, axis=0)
        m = jnp.mean(t2, axis=-1, keepdims=True)
        v = jnp.mean((t2 - m) ** 2, axis=-1, keepdims=True)
        t2 = (t2 - m) / jnp.sqrt(v + 1e-5) * lng[...] + lnb[...]
        h = h + t2

    rr = lax.broadcasted_iota(jnp.int32, (_H, _H), 0)
    cc = lax.broadcasted_iota(jnp.int32, (_H, _H), 1)
    eye = (rr == cc).astype(jnp.float32)
    for b in range(bsz):
        hb = h[b * _N:(b + 1) * _N]
        emb_ref[b] = hb
        embt_ref[b] = lax.dot_general(eye, hb, (((1,), (1,)), ((), ())),
                                      precision=lax.Precision.HIGHEST,
                                      preferred_element_type=jnp.float32)


def _gnn(bsz, wp, x, flat):
    def full(a):
        nd = a.ndim
        return pl.BlockSpec(a.shape, lambda g, _n=nd: (0,) * _n)

    tlast = x.shape[1] - 1
    in_specs = [full(wp),
                pl.BlockSpec((bsz, 1, _N, _F), lambda g: (0, tlast, 0, 0))]
    in_specs += [full(a) for a in flat]
    out_specs = [pl.BlockSpec((bsz, _N, _H), lambda g: (0, 0, 0)),
                 pl.BlockSpec((bsz, _H, _N), lambda g: (0, 0, 0))]
    return pl.pallas_call(
        functools.partial(_gnn_body, bsz),
        grid=(1,),
        in_specs=in_specs,
        out_specs=out_specs,
        out_shape=[jax.ShapeDtypeStruct((bsz, _N, _H), jnp.float32),
                   jax.ShapeDtypeStruct((bsz, _H, _N), jnp.float32)],
    )(wp, x, *flat)


def _dec_body(emb, embt, wa, wbt, o1b, olng, olnb, o2w, o2b, out_ref):
    e = emb[0]
    et = embt[0]
    a = _mm(e, wa[...]) + o1b[...]
    bt = _mm(wbt[...], et)
    gs = [olng[k] * o2w[k, 0] for k in range(_K)]
    gtot = 0.0
    cb = 0.0
    for k in range(_K):
        gtot = gtot + gs[k]
        cb = cb + olnb[k] * o2w[k, 0]
    cb = cb + o2b[0]
    for ic in range(_N // _BI):
        ac = a[ic * _BI:(ic + 1) * _BI]
        for jc in range(_N // _BJ):
            btc = bt[:, jc * _BJ:(jc + 1) * _BJ]
            s1 = jnp.zeros((_BI, _BJ), jnp.float32)
            s2 = jnp.zeros((_BI, _BJ), jnp.float32)
            sg = jnp.zeros((_BI, _BJ), jnp.float32)
            for k in range(_K):
                zk = jnp.maximum(ac[:, k:k + 1] + btc[k:k + 1, :], 0.0)
                s1 = s1 + zk
                s2 = s2 + zk * zk
                sg = sg + gs[k] * zk
            m = s1 * (1.0 / _K)
            v = s2 * (1.0 / _K) - m * m
            inv = 1.0 / jnp.sqrt(v + 1e-5)
            out_ref[0, ic * _BI:(ic + 1) * _BI, jc * _BJ:(jc + 1) * _BJ] = \
                jax.nn.sigmoid((sg - gtot * m) * inv + cb)


def _decode(bsz, emb, embt, wa, wbt, o1b, olng, olnb, o2w, o2b):
    smem = pltpu.MemorySpace.SMEM
    in_specs = [
        pl.BlockSpec((1, _N, _H), lambda b: (b, 0, 0)),
        pl.BlockSpec((1, _H, _N), lambda b: (b, 0, 0)),
        pl.BlockSpec((_H, _K), lambda b: (0, 0)),
        pl.BlockSpec((_K, _H), lambda b: (0, 0)),
        pl.BlockSpec((_K,), lambda b: (0,)),
        pl.BlockSpec(memory_space=smem),
        pl.BlockSpec(memory_space=smem),
        pl.BlockSpec(memory_space=smem),
        pl.BlockSpec(memory_space=smem),
    ]
    return pl.pallas_call(
        _dec_body,
        grid=(bsz,),
        in_specs=in_specs,
        out_specs=pl.BlockSpec((1, _N, _N), lambda b: (b, 0, 0)),
        out_shape=jax.ShapeDtypeStruct((bsz, _N, _N), jnp.float32),
    )(emb, embt, wa, wbt, o1b, olng, olnb, o2w, o2b)


def _flatten_params(p):
    flat = [p['in_w'], p['in_b']]
    for blk in p['blocks']:
        flat += [
            blk['t1c1_w'], blk['t1c1_b'],
            blk['t1c2_w'], blk['t1c2_b'],
            blk['t1c3_w'], blk['t1c3_b'],
            blk['cheb_w0'], blk['cheb_w1'], blk['cheb_b'],
            blk['t2c1_w'], blk['t2c1_b'],
            blk['t2c2_w'], blk['t2c2_b'],
            blk['t2c3_w'], blk['t2c3_b'],
            blk['bn_g'].reshape(_N, 1), blk['bn_b'].reshape(_N, 1),
            blk['ln_g'], blk['ln_b'],
        ]
    return flat


def kernel(x, edge_index, edge_weight, params):
    bsz = x.shape[0]
    wpart = _build_w(edge_index, edge_weight)
    wp = wpart.reshape(_N, _N)
    emb, embt = _gnn(bsz, wp, x, _flatten_params(params))
    p = params
    return _decode(
        bsz, emb, embt,
        p['o1_w'][:_H],
        p['o1_w'][_H:].T,
        p['o1_b'],
        p['oln_g'],
        p['oln_b'],
        p['o2_w'],
        p['o2_b'],
    )


# speedup vs baseline: 1.3851x; 1.0015x over previous
import functools
import math

import jax
import jax.numpy as jnp
from jax import lax
from jax.experimental import pallas as pl
from jax.experimental.pallas import tpu as pltpu
from jax.experimental.pallas import tpu_sc as plsc

_N = 512
_F = 64
_H = 64
_E = 8192
_NN = _N * _N
_K = 32
_BI = 128
_BJ = 128
_NBLK = 2

_EPT = _E // 16
_ZPT = _NN // 16


def _mm(a, b):
    dims = (((a.ndim - 1,), (0,)), ((), ()))
    return lax.dot_general(a, b, dims, preferred_element_type=jnp.float32)


def _sc_body(ei_hbm, ew_hbm, out_hbm, rowv, colv, wvv, idxb, valb, zbuf, accw,
             sem1, sem2, sem3):
    cid = lax.axis_index("c")
    sid = lax.axis_index("s")

    @pl.when(cid == 0)
    def _work():
        eb = sid * _EPT
        zb = sid * _ZPT
        st = [pltpu.async_copy(ei_hbm.at[0, pl.ds(eb, _EPT)], rowv, sem1),
              pltpu.async_copy(ei_hbm.at[1, pl.ds(eb, _EPT)], colv, sem1),
              pltpu.async_copy(ew_hbm.at[pl.ds(eb, _EPT)], wvv, sem1)]
        zv = jnp.zeros((16,), jnp.float32)
        for i in range(128):
            zbuf[pl.ds(i * 16, 16)] = zv
        zc = [pltpu.async_copy(zbuf, accw.at[pl.ds(zb + t * 2048, 2048)], sem2)
              for t in range(_ZPT // 2048)]
        for d in st:
            d.wait()
        for i in range(_EPT // 16):
            j, off = divmod(i, 8)
            c = colv[pl.ds(i * 16, 16)]
            r = rowv[pl.ds(i * 16, 16)]
            idxb[j, pl.ds(off * 16, 16)] = c * _N + r
            valb[j, pl.ds(off * 16, 16)] = wvv[pl.ds(i * 16, 16)]
        for d in zc:
            d.wait()
        plsc.subcore_barrier()
        sc = [pltpu.async_copy(valb.at[j], accw.at[idxb.at[j]], sem3, add=True)
              for j in range(_EPT // 128)]
        for d in sc:
            d.wait()
        plsc.subcore_barrier()
        pltpu.sync_copy(accw.at[pl.ds(zb, _ZPT)], out_hbm.at[pl.ds(zb, _ZPT)])


def _build_w(edge_index, edge_weight):
    mesh = plsc.VectorSubcoreMesh(core_axis_name="c", subcore_axis_name="s")
    return pl.kernel(
        _sc_body,
        out_type=jax.ShapeDtypeStruct((_NN,), jnp.float32),
        mesh=mesh,
        scratch_types=[
            pltpu.VMEM((_EPT,), jnp.int32),
            pltpu.VMEM((_EPT,), jnp.int32),
            pltpu.VMEM((_EPT,), jnp.float32),
            pltpu.VMEM((4, 128), jnp.int32),
            pltpu.VMEM((4, 128), jnp.float32),
            pltpu.VMEM((2048,), jnp.float32),
            pltpu.VMEM_SHARED((_NN,), jnp.float32),
            pltpu.SemaphoreType.DMA,
            pltpu.SemaphoreType.DMA,
            pltpu.SemaphoreType.DMA,
        ],
    )(edge_index, edge_weight)


def _tconv(h, w1, b1, w2, b2, w3, b3):
    p = _mm(h, w1[...]) + b1[...]
    q = jax.nn.sigmoid(_mm(h, w2[...]) + b2[...])
    r = _mm(h, w3[...]) + b3[...]
    return jnp.maximum(p * q + r, 0.0)


def _gnn_body(bsz, *refs):
    it = iter(refs)
    wp = next(it)
    x = next(it)
    inw = next(it)
    inb = next(it)
    blocks = []
    for _ in range(_NBLK):
        blocks.append([next(it) for _ in range(19)])
    emb_ref = next(it)
    embt_ref = next(it)

    wm = wp[...]
    ones_col = jnp.ones((_N, 1), jnp.float32)
    deg = lax.dot_general(wm, ones_col, (((0,), (0,)), ((), ())),
                          precision=lax.Precision.HIGHEST,
                          preferred_element_type=jnp.float32)
    pos = deg > 0.0
    dinv = jnp.where(pos, 1.0 / jnp.sqrt(jnp.where(pos, deg, 1.0)), 0.0)
    dinv_s = jnp.concatenate([dinv] * bsz, axis=0)

    xs = jnp.concatenate([x[b, 0] for b in range(bsz)], axis=0)
    h = _mm(xs, inw[...]) + inb[...]
    for blk in blocks:
        (t1w1, t1b1, t1w2, t1b2, t1w3, t1b3,
         cw0, cw1, cb,
         t2w1, t2b1, t2w2, t2b2, t2w3, t2b3,
         bng, bnb, lng, lnb) = blk
        t0 = _tconv(h, t1w1, t1b1, t1w2, t1b2, t1w3, t1b3)
        u = t0 * dinv_s
        ucat = jnp.concatenate(
            [u[b * _N:(b + 1) * _N] for b in range(bsz)], axis=1)
        ycat = _mm(wm, ucat)
        sx = -(dinv_s * jnp.concatenate(
            [ycat[:, b * _H:(b + 1) * _H] for b in range(bsz)], axis=0))
        t1 = jnp.maximum(_mm(t0, cw0[...]) + _mm(sx, cw1[...]) + cb[...], 0.0)
        t2 = _tconv(t1, t2w1, t2b1, t2w2, t2b2, t2w3, t2b3)
        bscale = bng[...] * (1.0 / math.sqrt(1.0 + 1e-5))
        t2 = t2 * jnp.concatenate([bscale] * bsz, axis=0) \
            + jnp.concatenate([bnb[...]] * bsz, axis=0)
        m = jnp.mean(t2, axis=-1, keepdims=True)
        v = jnp.mean((t2 - m) ** 2, axis=-1, keepdims=True)
        t2 = (t2 - m) / jnp.sqrt(v + 1e-5) * lng[...] + lnb[...]
        h = h + t2

    rr = lax.broadcasted_iota(jnp.int32, (_H, _H), 0)
    cc = lax.broadcasted_iota(jnp.int32, (_H, _H), 1)
    eye = (rr == cc).astype(jnp.float32)
    for b in range(bsz):
        hb = h[b * _N:(b + 1) * _N]
        emb_ref[b] = hb
        embt_ref[b] = lax.dot_general(eye, hb, (((1,), (1,)), ((), ())),
                                      precision=lax.Precision.HIGHEST,
                                      preferred_element_type=jnp.float32)


def _gnn(bsz, wp, x, flat):
    def full(a):
        nd = a.ndim
        return pl.BlockSpec(a.shape, lambda g, _n=nd: (0,) * _n)

    tlast = x.shape[1] - 1
    in_specs = [full(wp),
                pl.BlockSpec((bsz, 1, _N, _F), lambda g: (0, tlast, 0, 0))]
    in_specs += [full(a) for a in flat]
    out_specs = [pl.BlockSpec((bsz, _N, _H), lambda g: (0, 0, 0)),
                 pl.BlockSpec((bsz, _H, _N), lambda g: (0, 0, 0))]
    return pl.pallas_call(
        functools.partial(_gnn_body, bsz),
        grid=(1,),
        in_specs=in_specs,
        out_specs=out_specs,
        out_shape=[jax.ShapeDtypeStruct((bsz, _N, _H), jnp.float32),
                   jax.ShapeDtypeStruct((bsz, _H, _N), jnp.float32)],
    )(wp, x, *flat)


def _dec_body(emb, embt, wa, wbt, o1b, olng, olnb, o2w, o2b, out_ref):
    e = emb[0]
    et = embt[0]
    a = _mm(e, wa[...]) + o1b[...]
    bt = _mm(wbt[...], et)
    gs = [olng[k] * o2w[k, 0] for k in range(_K)]
    gtot = 0.0
    cb = 0.0
    for k in range(_K):
        gtot = gtot + gs[k]
        cb = cb + olnb[k] * o2w[k, 0]
    cb = cb + o2b[0]
    for ic in range(_N // _BI):
        ac = a[ic * _BI:(ic + 1) * _BI]
        for jc in range(_N // _BJ):
            btc = bt[:, jc * _BJ:(jc + 1) * _BJ]
            s1 = jnp.zeros((_BI, _BJ), jnp.float32)
            s2 = jnp.zeros((_BI, _BJ), jnp.float32)
            sg = jnp.zeros((_BI, _BJ), jnp.float32)
            for k in range(_K):
                zk = jnp.maximum(ac[:, k:k + 1] + btc[k:k + 1, :], 0.0)
                s1 = s1 + zk
                s2 = s2 + zk * zk
                sg = sg + gs[k] * zk
            m = s1 * (1.0 / _K)
            v = s2 * (1.0 / _K) - m * m
            inv = 1.0 / jnp.sqrt(v + 1e-5)
            out_ref[0, ic * _BI:(ic + 1) * _BI, jc * _BJ:(jc + 1) * _BJ] = \
                jax.nn.sigmoid((sg - gtot * m) * inv + cb)


def _decode(bsz, emb, embt, wa, wbt, o1b, olng, olnb, o2w, o2b):
    smem = pltpu.MemorySpace.SMEM
    in_specs = [
        pl.BlockSpec((1, _N, _H), lambda b: (b, 0, 0)),
        pl.BlockSpec((1, _H, _N), lambda b: (b, 0, 0)),
        pl.BlockSpec((_H, _K), lambda b: (0, 0)),
        pl.BlockSpec((_K, _H), lambda b: (0, 0)),
        pl.BlockSpec((_K,), lambda b: (0,)),
        pl.BlockSpec(memory_space=smem),
        pl.BlockSpec(memory_space=smem),
        pl.BlockSpec(memory_space=smem),
        pl.BlockSpec(memory_space=smem),
    ]
    return pl.pallas_call(
        _dec_body,
        grid=(bsz,),
        in_specs=in_specs,
        out_specs=pl.BlockSpec((1, _N, _N), lambda b: (b, 0, 0)),
        out_shape=jax.ShapeDtypeStruct((bsz, _N, _N), jnp.float32),
    )(emb, embt, wa, wbt, o1b, olng, olnb, o2w, o2b)


def _flatten_params(p):
    flat = [p['in_w'], p['in_b']]
    for blk in p['blocks']:
        flat += [
            blk['t1c1_w'], blk['t1c1_b'],
            blk['t1c2_w'], blk['t1c2_b'],
            blk['t1c3_w'], blk['t1c3_b'],
            blk['cheb_w0'], blk['cheb_w1'], blk['cheb_b'],
            blk['t2c1_w'], blk['t2c1_b'],
            blk['t2c2_w'], blk['t2c2_b'],
            blk['t2c3_w'], blk['t2c3_b'],
            blk['bn_g'].reshape(_N, 1), blk['bn_b'].reshape(_N, 1),
            blk['ln_g'], blk['ln_b'],
        ]
    return flat


def kernel(x, edge_index, edge_weight, params):
    bsz = x.shape[0]
    wpart = _build_w(edge_index, edge_weight)
    wp = wpart.reshape(_N, _N)
    emb, embt = _gnn(bsz, wp, x, _flatten_params(params))
    p = params
    return _decode(
        bsz, emb, embt,
        p['o1_w'][:_H],
        p['o1_w'][_H:].T,
        p['o1_b'],
        p['oln_g'],
        p['oln_b'],
        p['o2_w'],
        p['o2_b'],
    )


# speedup vs baseline: 1.5068x; 1.0879x over previous
import functools
import math

import jax
import jax.numpy as jnp
from jax import lax
from jax.experimental import pallas as pl
from jax.experimental.pallas import tpu as pltpu
from jax.experimental.pallas import tpu_sc as plsc

_N = 512
_F = 64
_H = 64
_E = 8192
_NN = _N * _N
_K = 32
_BI = 128
_BJ = 128
_NBLK = 2

_EPT = _E // 16
_ZPT = _NN // 16


def _mm(a, b):
    dims = (((a.ndim - 1,), (0,)), ((), ()))
    return lax.dot_general(a, b, dims, preferred_element_type=jnp.float32)


def _sc_body(ei_hbm, ew_hbm, out_hbm, rowv, colv, wvv, idxb, valb, zbuf, accw,
             sem1, sem2, sem3):
    cid = lax.axis_index("c")
    sid = lax.axis_index("s")

    @pl.when(cid == 0)
    def _work():
        eb = sid * _EPT
        zb = sid * _ZPT
        st = [pltpu.async_copy(ei_hbm.at[0, pl.ds(eb, _EPT)], rowv, sem1),
              pltpu.async_copy(ei_hbm.at[1, pl.ds(eb, _EPT)], colv, sem1),
              pltpu.async_copy(ew_hbm.at[pl.ds(eb, _EPT)], wvv, sem1)]
        zv = jnp.zeros((16,), jnp.float32)
        for i in range(128):
            zbuf[pl.ds(i * 16, 16)] = zv
        zc = [pltpu.async_copy(zbuf, accw.at[pl.ds(zb + t * 2048, 2048)], sem2)
              for t in range(_ZPT // 2048)]
        for d in st:
            d.wait()
        for i in range(_EPT // 16):
            j, off = divmod(i, 8)
            c = colv[pl.ds(i * 16, 16)]
            r = rowv[pl.ds(i * 16, 16)]
            idxb[j, pl.ds(off * 16, 16)] = c * _N + r
            valb[j, pl.ds(off * 16, 16)] = wvv[pl.ds(i * 16, 16)]
        for d in zc:
            d.wait()
        plsc.subcore_barrier()
        sc = [pltpu.async_copy(valb.at[j], accw.at[idxb.at[j]], sem3, add=True)
              for j in range(_EPT // 128)]
        for d in sc:
            d.wait()
        plsc.subcore_barrier()
        pltpu.sync_copy(accw.at[pl.ds(zb, _ZPT)], out_hbm.at[pl.ds(zb, _ZPT)])


def _build_w(edge_index, edge_weight):
    mesh = plsc.VectorSubcoreMesh(core_axis_name="c", subcore_axis_name="s")
    return pl.kernel(
        _sc_body,
        out_type=jax.ShapeDtypeStruct((_NN,), jnp.float32),
        mesh=mesh,
        scratch_types=[
            pltpu.VMEM((_EPT,), jnp.int32),
            pltpu.VMEM((_EPT,), jnp.int32),
            pltpu.VMEM((_EPT,), jnp.float32),
            pltpu.VMEM((4, 128), jnp.int32),
            pltpu.VMEM((4, 128), jnp.float32),
            pltpu.VMEM((2048,), jnp.float32),
            pltpu.VMEM_SHARED((_NN,), jnp.float32),
            pltpu.SemaphoreType.DMA,
            pltpu.SemaphoreType.DMA,
            pltpu.SemaphoreType.DMA,
        ],
    )(edge_index, edge_weight)


def _tconv(h, w1, b1, w2, b2, w3, b3):
    p = _mm(h, w1[...]) + b1[...]
    q = jax.nn.sigmoid(_mm(h, w2[...]) + b2[...])
    r = _mm(h, w3[...]) + b3[...]
    return jnp.maximum(p * q + r, 0.0)


def _gnn_body(bsz, *refs):
    it = iter(refs)
    wp = next(it)
    x = next(it)
    inw = next(it)
    inb = next(it)
    blocks = []
    for _ in range(_NBLK):
        blocks.append([next(it) for _ in range(19)])
    emb_ref = next(it)
    embt_ref = next(it)

    wm = wp[...]
    ones_col = jnp.ones((_N, 1), jnp.float32)
    deg = lax.dot_general(wm, ones_col, (((0,), (0,)), ((), ())),
                          precision=lax.Precision.HIGHEST,
                          preferred_element_type=jnp.float32)
    pos = deg > 0.0
    dinv = jnp.where(pos, 1.0 / jnp.sqrt(jnp.where(pos, deg, 1.0)), 0.0)
    dinv_s = jnp.concatenate([dinv] * bsz, axis=0)

    rrn = lax.broadcasted_iota(jnp.int32, (_N, _N), 0)
    ccn = lax.broadcasted_iota(jnp.int32, (_N, _N), 1)
    eyen = (rrn == ccn).astype(jnp.float32)

    def colify(v):
        return lax.dot_general(eyen, v.reshape(1, _N), (((1,), (1,)), ((), ())),
                               preferred_element_type=jnp.float32)

    xs = jnp.concatenate([x[b] for b in range(bsz)], axis=0)
    h = _mm(xs, inw[...]) + inb[...]
    for blk in blocks:
        (t1w1, t1b1, t1w2, t1b2, t1w3, t1b3,
         cw0, cw1, cb,
         t2w1, t2b1, t2w2, t2b2, t2w3, t2b3,
         bng, bnb, lng, lnb) = blk
        t0 = _tconv(h, t1w1, t1b1, t1w2, t1b2, t1w3, t1b3)
        u = t0 * dinv_s
        ucat = jnp.concatenate(
            [u[b * _N:(b + 1) * _N] for b in range(bsz)], axis=1)
        ycat = _mm(wm, ucat)
        sx = -(dinv_s * jnp.concatenate(
            [ycat[:, b * _H:(b + 1) * _H] for b in range(bsz)], axis=0))
        t1 = jnp.maximum(_mm(t0, cw0[...]) + _mm(sx, cw1[...]) + cb[...], 0.0)
        t2 = _tconv(t1, t2w1, t2b1, t2w2, t2b2, t2w3, t2b3)
        bscale = colify(bng[...] * (1.0 / math.sqrt(1.0 + 1e-5)))
        bshift = colify(bnb[...])
        t2 = t2 * jnp.concatenate([bscale] * bsz, axis=0) \
            + jnp.concatenate([bshift] * bsz, axis=0)
        m = jnp.mean(t2, axis=-1, keepdims=True)
        v = jnp.mean((t2 - m) ** 2, axis=-1, keepdims=True)
        t2 = (t2 - m) / jnp.sqrt(v + 1e-5) * lng[...] + lnb[...]
        h = h + t2

    rr = lax.broadcasted_iota(jnp.int32, (_H, _H), 0)
    cc = lax.broadcasted_iota(jnp.int32, (_H, _H), 1)
    eye = (rr == cc).astype(jnp.float32)
    for b in range(bsz):
        hb = h[b * _N:(b + 1) * _N]
        emb_ref[b] = hb
        embt_ref[b] = lax.dot_general(eye, hb, (((1,), (1,)), ((), ())),
                                      precision=lax.Precision.HIGHEST,
                                      preferred_element_type=jnp.float32)


def _gnn(bsz, wp, x, flat):
    def full(a):
        nd = a.ndim
        return pl.BlockSpec(a.shape, lambda g, _n=nd: (0,) * _n)

    in_specs = [full(wp), full(x)]
    in_specs += [full(a) for a in flat]
    out_specs = [pl.BlockSpec((bsz, _N, _H), lambda g: (0, 0, 0)),
                 pl.BlockSpec((bsz, _H, _N), lambda g: (0, 0, 0))]
    return pl.pallas_call(
        functools.partial(_gnn_body, bsz),
        grid=(1,),
        in_specs=in_specs,
        out_specs=out_specs,
        out_shape=[jax.ShapeDtypeStruct((bsz, _N, _H), jnp.float32),
                   jax.ShapeDtypeStruct((bsz, _H, _N), jnp.float32)],
    )(wp, x, *flat)


def _dec_body(emb, embt, o1w, o1b, olng, olnb, o2w, o2b, out_ref):
    e = emb[0]
    et = embt[0]
    rr = lax.broadcasted_iota(jnp.int32, (_K, _K), 0)
    cc = lax.broadcasted_iota(jnp.int32, (_K, _K), 1)
    eyek = (rr == cc).astype(jnp.float32)
    wbt = lax.dot_general(eyek, o1w[_H:, :], (((1,), (1,)), ((), ())),
                          preferred_element_type=jnp.float32)
    a = _mm(e, o1w[0:_H, :]) + o1b[...]
    bt = _mm(wbt, et)
    gs = [olng[k] * o2w[k, 0] for k in range(_K)]
    gtot = 0.0
    cb = 0.0
    for k in range(_K):
        gtot = gtot + gs[k]
        cb = cb + olnb[k] * o2w[k, 0]
    cb = cb + o2b[0]
    for ic in range(_N // _BI):
        ac = a[ic * _BI:(ic + 1) * _BI]
        for jc in range(_N // _BJ):
            btc = bt[:, jc * _BJ:(jc + 1) * _BJ]
            s1 = jnp.zeros((_BI, _BJ), jnp.float32)
            s2 = jnp.zeros((_BI, _BJ), jnp.float32)
            sg = jnp.zeros((_BI, _BJ), jnp.float32)
            for k in range(_K):
                zk = jnp.maximum(ac[:, k:k + 1] + btc[k:k + 1, :], 0.0)
                s1 = s1 + zk
                s2 = s2 + zk * zk
                sg = sg + gs[k] * zk
            m = s1 * (1.0 / _K)
            v = s2 * (1.0 / _K) - m * m
            inv = 1.0 / jnp.sqrt(v + 1e-5)
            out_ref[0, ic * _BI:(ic + 1) * _BI, jc * _BJ:(jc + 1) * _BJ] = \
                jax.nn.sigmoid((sg - gtot * m) * inv + cb)


def _decode(bsz, emb, embt, o1w, o1b, olng, olnb, o2w, o2b):
    smem = pltpu.MemorySpace.SMEM
    in_specs = [
        pl.BlockSpec((1, _N, _H), lambda b: (b, 0, 0)),
        pl.BlockSpec((1, _H, _N), lambda b: (b, 0, 0)),
        pl.BlockSpec((2 * _H, _K), lambda b: (0, 0)),
        pl.BlockSpec((_K,), lambda b: (0,)),
        pl.BlockSpec(memory_space=smem),
        pl.BlockSpec(memory_space=smem),
        pl.BlockSpec(memory_space=smem),
        pl.BlockSpec(memory_space=smem),
    ]
    return pl.pallas_call(
        _dec_body,
        grid=(bsz,),
        in_specs=in_specs,
        out_specs=pl.BlockSpec((1, _N, _N), lambda b: (b, 0, 0)),
        out_shape=jax.ShapeDtypeStruct((bsz, _N, _N), jnp.float32),
    )(emb, embt, o1w, o1b, olng, olnb, o2w, o2b)


def _flatten_params(p):
    flat = [p['in_w'], p['in_b']]
    for blk in p['blocks']:
        flat += [
            blk['t1c1_w'], blk['t1c1_b'],
            blk['t1c2_w'], blk['t1c2_b'],
            blk['t1c3_w'], blk['t1c3_b'],
            blk['cheb_w0'], blk['cheb_w1'], blk['cheb_b'],
            blk['t2c1_w'], blk['t2c1_b'],
            blk['t2c2_w'], blk['t2c2_b'],
            blk['t2c3_w'], blk['t2c3_b'],
            blk['bn_g'], blk['bn_b'],
            blk['ln_g'], blk['ln_b'],
        ]
    return flat


def kernel(x, edge_index, edge_weight, params):
    bsz = x.shape[0]
    wpart = _build_w(edge_index, edge_weight)
    wp = wpart.reshape(_N, _N)
    xe = x[:, -1]
    emb, embt = _gnn(bsz, wp, xe, _flatten_params(params))
    p = params
    return _decode(
        bsz, emb, embt,
        p['o1_w'],
        p['o1_b'],
        p['oln_g'],
        p['oln_b'],
        p['o2_w'],
        p['o2_b'],
    )


# speedup vs baseline: 1.5583x; 1.0342x over previous
import functools
import math

import jax
import jax.numpy as jnp
from jax import lax
from jax.experimental import pallas as pl
from jax.experimental.pallas import tpu as pltpu
from jax.experimental.pallas import tpu_sc as plsc

_N = 512
_F = 64
_H = 64
_E = 8192
_NN = _N * _N
_K = 32
_BI = 128
_BJ = 128
_NBLK = 2

_EPT = _E // 16
_ZPT = _NN // 16


def _mm(a, b):
    dims = (((a.ndim - 1,), (0,)), ((), ()))
    return lax.dot_general(a, b, dims, preferred_element_type=jnp.float32)


def _sc_body(ei_hbm, ew_hbm, out_hbm, rowv, colv, wvv, idxb, valb, zbuf, accw,
             sem1, sem2, sem3):
    cid = lax.axis_index("c")
    sid = lax.axis_index("s")

    @pl.when(cid == 0)
    def _work():
        eb = sid * _EPT
        zb = sid * _ZPT
        st = [pltpu.async_copy(ei_hbm.at[0, pl.ds(eb, _EPT)], rowv, sem1),
              pltpu.async_copy(ei_hbm.at[1, pl.ds(eb, _EPT)], colv, sem1),
              pltpu.async_copy(ew_hbm.at[pl.ds(eb, _EPT)], wvv, sem1)]
        zv = jnp.zeros((16,), jnp.float32)
        for i in range(128):
            zbuf[pl.ds(i * 16, 16)] = zv
        zc = [pltpu.async_copy(zbuf, accw.at[pl.ds(zb + t * 2048, 2048)], sem2)
              for t in range(_ZPT // 2048)]
        for d in st:
            d.wait()
        for i in range(_EPT // 16):
            j, off = divmod(i, 8)
            c = colv[pl.ds(i * 16, 16)]
            r = rowv[pl.ds(i * 16, 16)]
            idxb[j, pl.ds(off * 16, 16)] = c * _N + r
            valb[j, pl.ds(off * 16, 16)] = wvv[pl.ds(i * 16, 16)]
        for d in zc:
            d.wait()
        plsc.subcore_barrier()
        sc = [pltpu.async_copy(valb.at[j], accw.at[idxb.at[j]], sem3, add=True)
              for j in range(_EPT // 128)]
        for d in sc:
            d.wait()
        plsc.subcore_barrier()
        pltpu.sync_copy(accw.at[pl.ds(zb, _ZPT)], out_hbm.at[0, pl.ds(zb, _ZPT)])


def _build_w(edge_index, edge_weight):
    mesh = plsc.VectorSubcoreMesh(core_axis_name="c", subcore_axis_name="s")
    return pl.kernel(
        _sc_body,
        out_type=jax.ShapeDtypeStruct((1, _NN), jnp.float32),
        mesh=mesh,
        scratch_types=[
            pltpu.VMEM((_EPT,), jnp.int32),
            pltpu.VMEM((_EPT,), jnp.int32),
            pltpu.VMEM((_EPT,), jnp.float32),
            pltpu.VMEM((4, 128), jnp.int32),
            pltpu.VMEM((4, 128), jnp.float32),
            pltpu.VMEM((2048,), jnp.float32),
            pltpu.VMEM_SHARED((_NN,), jnp.float32),
            pltpu.SemaphoreType.DMA,
            pltpu.SemaphoreType.DMA,
            pltpu.SemaphoreType.DMA,
        ],
    )(edge_index, edge_weight)


def _tconv(h, w1, b1, w2, b2, w3, b3):
    p = _mm(h, w1[...]) + b1[...]
    q = jax.nn.sigmoid(_mm(h, w2[...]) + b2[...])
    r = _mm(h, w3[...]) + b3[...]
    return jnp.maximum(p * q + r, 0.0)


def _gnn_body(bsz, *refs):
    it = iter(refs)
    wp_hbm = next(it)
    x = next(it)
    inw = next(it)
    inb = next(it)
    blocks = []
    for _ in range(_NBLK):
        blocks.append([next(it) for _ in range(19)])
    emb_ref = next(it)
    embt_ref = next(it)
    wms = next(it)
    wsem = next(it)

    dma = pltpu.async_copy(wp_hbm.reshape(_N, _N), wms, wsem)

    rrn = lax.broadcasted_iota(jnp.int32, (_N, _N), 0)
    ccn = lax.broadcasted_iota(jnp.int32, (_N, _N), 1)
    eyen = (rrn == ccn).astype(jnp.float32)

    def colify(v):
        return lax.dot_general(eyen, v.reshape(1, _N), (((1,), (1,)), ((), ())),
                               preferred_element_type=jnp.float32)

    xs = jnp.concatenate([x[b] for b in range(bsz)], axis=0)
    h = _mm(xs, inw[...]) + inb[...]
    first = blocks[0]
    t0 = _tconv(h, first[0], first[1], first[2], first[3], first[4], first[5])

    dma.wait()
    wm = wms[...]
    ones_col = jnp.ones((_N, 1), jnp.float32)
    deg = lax.dot_general(wm, ones_col, (((0,), (0,)), ((), ())),
                          precision=lax.Precision.HIGHEST,
                          preferred_element_type=jnp.float32)
    pos = deg > 0.0
    dinv = jnp.where(pos, 1.0 / jnp.sqrt(jnp.where(pos, deg, 1.0)), 0.0)
    dinv_s = jnp.concatenate([dinv] * bsz, axis=0)

    for bi, blk in enumerate(blocks):
        (t1w1, t1b1, t1w2, t1b2, t1w3, t1b3,
         cw0, cw1, cb,
         t2w1, t2b1, t2w2, t2b2, t2w3, t2b3,
         bng, bnb, lng, lnb) = blk
        if bi > 0:
            t0 = _tconv(h, t1w1, t1b1, t1w2, t1b2, t1w3, t1b3)
        u = t0 * dinv_s
        ucat = jnp.concatenate(
            [u[b * _N:(b + 1) * _N] for b in range(bsz)], axis=1)
        ycat = _mm(wm, ucat)
        sx = -(dinv_s * jnp.concatenate(
            [ycat[:, b * _H:(b + 1) * _H] for b in range(bsz)], axis=0))
        t1 = jnp.maximum(_mm(t0, cw0[...]) + _mm(sx, cw1[...]) + cb[...], 0.0)
        t2 = _tconv(t1, t2w1, t2b1, t2w2, t2b2, t2w3, t2b3)
        bscale = colify(bng[...] * (1.0 / math.sqrt(1.0 + 1e-5)))
        bshift = colify(bnb[...])
        t2 = t2 * jnp.concatenate([bscale] * bsz, axis=0) \
            + jnp.concatenate([bshift] * bsz, axis=0)
        m = jnp.mean(t2, axis=-1, keepdims=True)
        v = jnp.mean((t2 - m) ** 2, axis=-1, keepdims=True)
        t2 = (t2 - m) / jnp.sqrt(v + 1e-5) * lng[...] + lnb[...]
        h = h + t2

    rr = lax.broadcasted_iota(jnp.int32, (_H, _H), 0)
    cc = lax.broadcasted_iota(jnp.int32, (_H, _H), 1)
    eye = (rr == cc).astype(jnp.float32)
    for b in range(bsz):
        hb = h[b * _N:(b + 1) * _N]
        emb_ref[b] = hb
        embt_ref[b] = lax.dot_general(eye, hb, (((1,), (1,)), ((), ())),
                                      precision=lax.Precision.HIGHEST,
                                      preferred_element_type=jnp.float32)


def _gnn(bsz, wp, x, flat):
    def full(a):
        nd = a.ndim
        return pl.BlockSpec(a.shape, lambda g, _n=nd: (0,) * _n)

    in_specs = [pl.BlockSpec(memory_space=pltpu.MemorySpace.HBM), full(x)]
    in_specs += [full(a) for a in flat]
    out_specs = [pl.BlockSpec((bsz, _N, _H), lambda g: (0, 0, 0)),
                 pl.BlockSpec((bsz, _H, _N), lambda g: (0, 0, 0))]
    return pl.pallas_call(
        functools.partial(_gnn_body, bsz),
        grid=(1,),
        in_specs=in_specs,
        out_specs=out_specs,
        out_shape=[jax.ShapeDtypeStruct((bsz, _N, _H), jnp.float32),
                   jax.ShapeDtypeStruct((bsz, _H, _N), jnp.float32)],
        scratch_shapes=[pltpu.VMEM((_N, _N), jnp.float32),
                        pltpu.SemaphoreType.DMA],
    )(wp, x, *flat)


def _dec_body(emb, embt, o1w, o1b, olng, olnb, o2w, o2b, out_ref):
    e = emb[0]
    et = embt[0]
    rr = lax.broadcasted_iota(jnp.int32, (_K, _K), 0)
    cc = lax.broadcasted_iota(jnp.int32, (_K, _K), 1)
    eyek = (rr == cc).astype(jnp.float32)
    wbt = lax.dot_general(eyek, o1w[_H:, :], (((1,), (1,)), ((), ())),
                          preferred_element_type=jnp.float32)
    a = _mm(e, o1w[0:_H, :]) + o1b[...]
    bt = _mm(wbt, et)
    gs = [olng[k] * o2w[k, 0] for k in range(_K)]
    gtot = 0.0
    cb = 0.0
    for k in range(_K):
        gtot = gtot + gs[k]
        cb = cb + olnb[k] * o2w[k, 0]
    cb = cb + o2b[0]
    for ic in range(_N // _BI):
        ac = a[ic * _BI:(ic + 1) * _BI]
        for jc in range(_N // _BJ):
            btc = bt[:, jc * _BJ:(jc + 1) * _BJ]
            s1 = jnp.zeros((_BI, _BJ), jnp.float32)
            s2 = jnp.zeros((_BI, _BJ), jnp.float32)
            sg = jnp.zeros((_BI, _BJ), jnp.float32)
            for k in range(_K):
                zk = jnp.maximum(ac[:, k:k + 1] + btc[k:k + 1, :], 0.0)
                s1 = s1 + zk
                s2 = s2 + zk * zk
                sg = sg + gs[k] * zk
            m = s1 * (1.0 / _K)
            v = s2 * (1.0 / _K) - m * m
            inv = 1.0 / jnp.sqrt(v + 1e-5)
            out_ref[0, ic * _BI:(ic + 1) * _BI, jc * _BJ:(jc + 1) * _BJ] = \
                jax.nn.sigmoid((sg - gtot * m) * inv + cb)


def _decode(bsz, emb, embt, o1w, o1b, olng, olnb, o2w, o2b):
    smem = pltpu.MemorySpace.SMEM
    in_specs = [
        pl.BlockSpec((1, _N, _H), lambda b: (b, 0, 0)),
        pl.BlockSpec((1, _H, _N), lambda b: (b, 0, 0)),
        pl.BlockSpec((2 * _H, _K), lambda b: (0, 0)),
        pl.BlockSpec((_K,), lambda b: (0,)),
        pl.BlockSpec(memory_space=smem),
        pl.BlockSpec(memory_space=smem),
        pl.BlockSpec(memory_space=smem),
        pl.BlockSpec(memory_space=smem),
    ]
    return pl.pallas_call(
        _dec_body,
        grid=(bsz,),
        in_specs=in_specs,
        out_specs=pl.BlockSpec((1, _N, _N), lambda b: (b, 0, 0)),
        out_shape=jax.ShapeDtypeStruct((bsz, _N, _N), jnp.float32),
    )(emb, embt, o1w, o1b, olng, olnb, o2w, o2b)


def _flatten_params(p):
    flat = [p['in_w'], p['in_b']]
    for blk in p['blocks']:
        flat += [
            blk['t1c1_w'], blk['t1c1_b'],
            blk['t1c2_w'], blk['t1c2_b'],
            blk['t1c3_w'], blk['t1c3_b'],
            blk['cheb_w0'], blk['cheb_w1'], blk['cheb_b'],
            blk['t2c1_w'], blk['t2c1_b'],
            blk['t2c2_w'], blk['t2c2_b'],
            blk['t2c3_w'], blk['t2c3_b'],
            blk['bn_g'], blk['bn_b'],
            blk['ln_g'], blk['ln_b'],
        ]
    return flat


def kernel(x, edge_index, edge_weight, params):
    bsz = x.shape[0]
    wpart = _build_w(edge_index, edge_weight)
    xe = x[:, -1]
    emb, embt = _gnn(bsz, wpart, xe, _flatten_params(params))
    p = params
    return _decode(
        bsz, emb, embt,
        p['o1_w'],
        p['o1_b'],
        p['oln_g'],
        p['oln_b'],
        p['o2_w'],
        p['o2_b'],
    )
